# baseline (speedup 1.0000x reference)
import jax
import jax.numpy as jnp
from jax.experimental import pallas as pl
from jax.experimental.pallas import tpu as pltpu

_BIG = 8.0


def _daa_body(etc_ref, g_ref, x_ref, out_ref, a0_ref, a1_ref, c_ref):
    out_feats, in_feats = a0_ref.shape
    bn = x_ref.shape[0]

    @pl.when(pl.program_id(0) == 0)
    def _compute_routing():
        v0 = etc_ref[0] + g_ref[0]
        v1 = etc_ref[1] + g_ref[1]
        v2 = etc_ref[2] + g_ref[2]
        m0 = (v0 >= v1) & (v0 >= v2)
        m1 = jnp.logical_not(m0) & (v1 >= v2)
        a0_ref[...] = jnp.where(m0, 0.0, _BIG).astype(jnp.bfloat16)
        a1_ref[...] = jnp.where(m1, 0.0, _BIG).astype(jnp.bfloat16)
        any_edge = jnp.any(m0 | m1, axis=1)
        rows1 = jax.lax.broadcasted_iota(jnp.int32, (out_feats, 1), 0)
        neutral = jnp.where(rows1 % 2 == 0, 2.0, 1.0)[:, 0]
        c_ref[...] = jnp.where(any_edge, _BIG, neutral)[None, :]

    xb = x_ref[...]
    p = xb.astype(jnp.bfloat16)
    q = (1.0 - xb).astype(jnp.bfloat16)
    pn = -p
    qn = -q
    for node in range(out_feats):
        pa, qa = (p, q) if node % 2 == 0 else (pn, qn)
        t = jnp.minimum(pa + a0_ref[node, :][None, :],
                        qa + a1_ref[node, :][None, :])
        m = jnp.min(t, axis=1).astype(jnp.float32)
        m = jnp.minimum(m, c_ref[0, node])
        out_ref[:, node] = m if node % 2 == 0 else -m


def kernel(x, edge_type_count):
    n, in_feats = x.shape
    out_feats = edge_type_count.shape[0]
    u = jax.random.uniform(jax.random.key(42), edge_type_count.shape,
                           minval=1e-6, maxval=1.0 - 1e-6)
    g = -jnp.log(-jnp.log(u))
    etc_t = jnp.transpose(edge_type_count, (2, 0, 1))
    g_t = jnp.transpose(g, (2, 0, 1))

    bn = 1024
    grid = (n // bn,)
    return pl.pallas_call(
        _daa_body,
        grid=grid,
        in_specs=[
            pl.BlockSpec((3, out_feats, in_feats), lambda i: (0, 0, 0)),
            pl.BlockSpec((3, out_feats, in_feats), lambda i: (0, 0, 0)),
            pl.BlockSpec((bn, in_feats), lambda i: (i, 0)),
        ],
        out_specs=pl.BlockSpec((bn, out_feats), lambda i: (i, 0)),
        out_shape=jax.ShapeDtypeStruct((n, out_feats), jnp.float32),
        scratch_shapes=[
            pltpu.VMEM((out_feats, in_feats), jnp.bfloat16),
            pltpu.VMEM((out_feats, in_feats), jnp.bfloat16),
            pltpu.VMEM((1, out_feats), jnp.float32),
        ],
    )(etc_t, g_t, x)


# speedup vs baseline: 1.1763x; 1.1763x over previous
import jax
import jax.numpy as jnp
from jax.experimental import pallas as pl
from jax.experimental.pallas import tpu as pltpu

_BIG = 8.0


def _daa_body(etc_ref, g_ref, x_ref, out_ref, a0_ref, a1_ref, c_ref):
    out_feats, in_feats = a0_ref.shape
    bn = x_ref.shape[0]

    @pl.when(pl.program_id(0) == 0)
    def _compute_routing():
        v0 = etc_ref[0] + g_ref[0]
        v1 = etc_ref[1] + g_ref[1]
        v2 = etc_ref[2] + g_ref[2]
        m0 = (v0 >= v1) & (v0 >= v2)
        m1 = jnp.logical_not(m0) & (v1 >= v2)
        a0_ref[...] = jnp.where(m0, 0.0, _BIG).astype(jnp.bfloat16)
        a1_ref[...] = jnp.where(m1, 0.0, _BIG).astype(jnp.bfloat16)
        any_edge = jnp.any(m0 | m1, axis=1)
        rows1 = jax.lax.broadcasted_iota(jnp.int32, (out_feats, 1), 0)
        neutral = jnp.where(rows1 % 2 == 0, 2.0, 1.0)[:, 0]
        c_ref[...] = jnp.where(any_edge, _BIG, neutral)[None, :]

    xb = x_ref[...]
    p = xb.astype(jnp.bfloat16)
    q = (1.0 - xb).astype(jnp.bfloat16)
    pn = -p
    qn = -q
    mins = []
    for node in range(out_feats):
        pa, qa = (p, q) if node % 2 == 0 else (pn, qn)
        t = jnp.minimum(pa + a0_ref[node, :][None, :],
                        qa + a1_ref[node, :][None, :])
        mins.append(jnp.min(t, axis=1))
    m = jnp.stack(mins, axis=1).astype(jnp.float32)
    m = jnp.minimum(m, c_ref[...])
    cols = jax.lax.broadcasted_iota(jnp.int32, (bn, out_feats), 1)
    sgn = jnp.where(cols % 2 == 0, 1.0, -1.0)
    out_ref[...] = m * sgn


def kernel(x, edge_type_count):
    n, in_feats = x.shape
    out_feats = edge_type_count.shape[0]
    u = jax.random.uniform(jax.random.key(42), edge_type_count.shape,
                           minval=1e-6, maxval=1.0 - 1e-6)
    g = -jnp.log(-jnp.log(u))
    etc_t = jnp.transpose(edge_type_count, (2, 0, 1))
    g_t = jnp.transpose(g, (2, 0, 1))

    bn = 1024
    grid = (n // bn,)
    return pl.pallas_call(
        _daa_body,
        grid=grid,
        in_specs=[
            pl.BlockSpec((3, out_feats, in_feats), lambda i: (0, 0, 0)),
            pl.BlockSpec((3, out_feats, in_feats), lambda i: (0, 0, 0)),
            pl.BlockSpec((bn, in_feats), lambda i: (i, 0)),
        ],
        out_specs=pl.BlockSpec((bn, out_feats), lambda i: (i, 0)),
        out_shape=jax.ShapeDtypeStruct((n, out_feats), jnp.float32),
        scratch_shapes=[
            pltpu.VMEM((out_feats, in_feats), jnp.bfloat16),
            pltpu.VMEM((out_feats, in_feats), jnp.bfloat16),
            pltpu.VMEM((1, out_feats), jnp.float32),
        ],
    )(etc_t, g_t, x)
